# Initial kernel scaffold; baseline (speedup 1.0000x reference)
import jax
import jax.numpy as jnp
from jax.experimental import pallas as pl


def kernel(scores, boxes, classes):
    raise NotImplementedError("write your pallas kernel here")


# speedup vs baseline: 4.4583x; 4.4583x over previous
import functools

import jax
import jax.numpy as jnp
from jax.experimental import pallas as pl
from jax.experimental.pallas import tpu as pltpu

_IOU_THRESHOLD = 0.5
_K = 300
_INT32_MAX = 2147483647
_INTERPRET = False


def _nms_body(scores_ref, x1_ref, y1_ref, x2_ref, y2_ref, cls_ref,
              sel_ref, osc_ref, ox1_ref, oy1_ref, ox2_ref, oy2_ref,
              ocl_ref, cnt_ref, ms_ref, n_valid: int):
    B, Np = scores_ref.shape
    scores = scores_ref[...]
    x1 = x1_ref[...]
    y1 = y1_ref[...]
    x2 = x2_ref[...]
    y2 = y2_ref[...]
    cls_f = cls_ref[...]
    areas = (x2 - x1) * (y2 - y1)
    iota = jax.lax.broadcasted_iota(jnp.int32, (B, Np), 1)
    neg_inf = jnp.float32(-jnp.inf)

    ms_ref[...] = jnp.where(iota < n_valid, scores, neg_inf)

    def body(i, count):
        ms = ms_ref[...]
        m = jnp.max(ms, axis=1, keepdims=True)
        pick = m > neg_inf
        eqm = ms == m
        idx = jnp.min(jnp.where(eqm, iota, jnp.int32(_INT32_MAX)),
                      axis=1, keepdims=True)
        eq1 = iota == idx
        zf = jnp.float32(0.0)
        bx1 = jnp.sum(jnp.where(eq1, x1, zf), axis=1, keepdims=True)
        by1 = jnp.sum(jnp.where(eq1, y1, zf), axis=1, keepdims=True)
        bx2 = jnp.sum(jnp.where(eq1, x2, zf), axis=1, keepdims=True)
        by2 = jnp.sum(jnp.where(eq1, y2, zf), axis=1, keepdims=True)
        bsc = jnp.sum(jnp.where(eq1, scores, zf), axis=1, keepdims=True)
        bcl = jnp.sum(jnp.where(eq1, cls_f, zf), axis=1, keepdims=True)
        barea = (bx2 - bx1) * (by2 - by1)
        xx1 = jnp.maximum(x1, bx1)
        yy1 = jnp.maximum(y1, by1)
        xx2 = jnp.minimum(x2, bx2)
        yy2 = jnp.minimum(y2, by2)
        inter = jnp.maximum(xx2 - xx1, 0.0) * jnp.maximum(yy2 - yy1, 0.0)
        iou = inter / (areas + barea - inter + 1e-9)
        keep_col = jnp.where(pick, idx, jnp.int32(-1))
        suppress = ((iou > _IOU_THRESHOLD) & pick) | eq1
        ms_ref[...] = jnp.where(suppress, neg_inf, ms)
        count = count + pick.astype(jnp.int32)

        sel_ref[pl.ds(i, 1)] = keep_col[None]
        osc_ref[pl.ds(i, 1)] = bsc[None]
        ox1_ref[pl.ds(i, 1)] = bx1[None]
        oy1_ref[pl.ds(i, 1)] = by1[None]
        ox2_ref[pl.ds(i, 1)] = bx2[None]
        oy2_ref[pl.ds(i, 1)] = by2[None]
        ocl_ref[pl.ds(i, 1)] = bcl[None]

        return count

    count0 = jnp.zeros((B, 1), jnp.int32)
    count = jax.lax.fori_loop(0, _K + 1, body, count0)
    cnt_ref[:, :] = jnp.broadcast_to(count, cnt_ref.shape)


def kernel(scores, boxes, classes):
    B, N = scores.shape
    Np = ((N + 127) // 128) * 128
    pad = Np - N
    scores_p = jnp.pad(scores, ((0, 0), (0, pad)))
    x1 = jnp.pad(boxes[:, :, 0], ((0, 0), (0, pad)))
    y1 = jnp.pad(boxes[:, :, 1], ((0, 0), (0, pad)))
    x2 = jnp.pad(boxes[:, :, 2], ((0, 0), (0, pad)))
    y2 = jnp.pad(boxes[:, :, 3], ((0, 0), (0, pad)))
    cls_f = jnp.pad(classes.astype(jnp.float32), ((0, 0), (0, pad)))

    K1 = _K + 1
    out_shapes = [
        jax.ShapeDtypeStruct((K1, B, 1), jnp.int32),
        jax.ShapeDtypeStruct((K1, B, 1), jnp.float32),
        jax.ShapeDtypeStruct((K1, B, 1), jnp.float32),
        jax.ShapeDtypeStruct((K1, B, 1), jnp.float32),
        jax.ShapeDtypeStruct((K1, B, 1), jnp.float32),
        jax.ShapeDtypeStruct((K1, B, 1), jnp.float32),
        jax.ShapeDtypeStruct((K1, B, 1), jnp.float32),
        jax.ShapeDtypeStruct((B, 128), jnp.int32),
    ]
    outs = pl.pallas_call(
        functools.partial(_nms_body, n_valid=N),
        out_shape=out_shapes,
        scratch_shapes=[pltpu.VMEM((B, Np), jnp.float32)],
        interpret=_INTERPRET,
    )(scores_p, x1, y1, x2, y2, cls_f)

    sel_p, osc, ox1, oy1, ox2, oy2, ocl, cnt = (
        o[:_K, :, 0].T if o.ndim == 3 else o for o in outs)
    sel = sel_p
    count = cnt[:, 0]
    overflow = count > _K
    count = jnp.minimum(count, _K)
    eff = jnp.where(overflow, jnp.int32(_K - 1), count)
    m = jnp.arange(_K, dtype=jnp.int32)[None, :] < eff[:, None]
    out_scores = jnp.where(m, osc, 0.0)
    out_boxes = jnp.where(
        m[:, :, None],
        jnp.stack([ox1, oy1, ox2, oy2], axis=-1),
        0.0)
    out_classes = jnp.where(m, ocl.astype(jnp.int32), jnp.int32(_INT32_MAX))
    true_max = jnp.where(overflow, jnp.int32(-1), count).astype(jnp.int32)
    return (sel, out_scores, out_boxes, out_classes, true_max)


# speedup vs baseline: 6.8825x; 1.5438x over previous
import functools

import jax
import jax.numpy as jnp
from jax import lax
from jax.experimental import pallas as pl
from jax.experimental.pallas import tpu as pltpu
from jax.experimental.pallas import tpu_sc as plsc

_IOU_THRESHOLD = 0.5
_K = 300
_K1 = _K + 1
_KP = 320
_INT32_MAX = 2147483647
_CH = 1024
_ROWS = _CH // 128


def _take16(v, idx):
    dnums = lax.GatherDimensionNumbers(
        offset_dims=(), collapsed_slice_dims=(0,), start_index_map=(0,))
    return lax.gather(v, idx[:, None], dnums, slice_sizes=(1,),
                      mode=lax.GatherScatterMode.PROMISE_IN_BOUNDS)


def _nms_sc_body(n_boxes, np_pad, n_batches,
                 sc_hbm, ord_hbm, x1_hbm, y1_hbm, x2_hbm, y2_hbm, cls_hbm,
                 sel_o, osc_o, ox1_o, oy1_o, ox2_o, oy2_o, ocl_o, meta_o,
                 idx_v, sc_v, x1_v, y1_v, x2_v, y2_v, cls_v,
                 kx1, ky1, kx2, ky2, kar, ksel, ksc, kcl, meta_v, bad_v,
                 sem):
    cid = lax.axis_index("c")
    sid = lax.axis_index("s")
    b = sid * 2 + cid
    i16 = lax.broadcasted_iota(jnp.int32, (16,), 0)
    lane0 = i16 == 0

    @pl.when(b < n_batches)
    def _run():
        zf = jnp.zeros((16,), jnp.float32)
        neg1 = jnp.full((16,), -1, jnp.int32)
        zi = jnp.zeros((16,), jnp.int32)
        for j in range(_KP // 16):
            s = pl.ds(j * 16, 16)
            kx1[s] = zf
            ky1[s] = zf
            kx2[s] = zf
            ky2[s] = zf
            kar[s] = zf
            ksel[s] = neg1
            ksc[s] = zf
            kcl[s] = zi

        nchunks = (n_boxes + _CH - 1) // _CH
        flat_base = b * n_boxes
        pad_base = b * np_pad

        def chunk_body(ci, count):
            chunk_active = count < _K1
            c0 = ci * _CH
            clen = jnp.minimum(n_boxes - c0, _CH)

            @pl.when(chunk_active)
            def _fetch():
                pltpu.sync_copy(sc_hbm.at[pl.ds(pad_base + c0, _CH)],
                                sc_v.at[pl.ds(0, _CH)])
                pltpu.sync_copy(ord_hbm.at[pl.ds(pad_base + c0, _CH)],
                                idx_v.at[pl.ds(0, _CH)])
                for j in range(_ROWS):
                    row = idx_v.at[pl.ds(j * 128, 128)]
                    dst = pl.ds(j * 128, 128)
                    cps = (pltpu.async_copy(x1_hbm.at[row], x1_v.at[dst], sem),
                           pltpu.async_copy(y1_hbm.at[row], y1_v.at[dst], sem),
                           pltpu.async_copy(x2_hbm.at[row], x2_v.at[dst], sem),
                           pltpu.async_copy(y2_hbm.at[row], y2_v.at[dst], sem),
                           pltpu.async_copy(cls_hbm.at[row], cls_v.at[dst],
                                            sem))
                    for cp in cps:
                        cp.wait()

            def cand_body(i, count2):
                active = count2 < _K1
                iw = pl.ds(i, 16)
                x1c = x1_v[iw][0]
                y1c = y1_v[iw][0]
                x2c = x2_v[iw][0]
                y2c = y2_v[iw][0]
                ac = (x2c - x1c) * (y2c - y1c)

                nkc = jnp.where(active, (count2 + 15) // 16, 0)
                bad_v[...] = jnp.zeros((16,), jnp.int32)

                def kchunk(jk, tok):
                    ks = pl.ds(jk * 16, 16)
                    xx1 = jnp.maximum(kx1[ks], x1c)
                    yy1 = jnp.maximum(ky1[ks], y1c)
                    xx2 = jnp.minimum(kx2[ks], x2c)
                    yy2 = jnp.minimum(ky2[ks], y2c)
                    inter = (jnp.maximum(xx2 - xx1, 0.0) *
                             jnp.maximum(yy2 - yy1, 0.0))
                    iou = inter / (kar[ks] + ac - inter + 1e-9)
                    bad_v[...] = bad_v[...] | jnp.where(
                        iou > _IOU_THRESHOLD, 1, 0)
                    return tok

                lax.fori_loop(0, nkc, kchunk, jnp.int32(0))
                v = bad_v[...]
                for sh in (8, 4, 2, 1):
                    v = jnp.maximum(v, _take16(v, (i16 + sh) & 15))
                keep = active & (v[0] == 0)

                @pl.when(keep & (count2 < _K))
                def _store():
                    slot = pl.ds(count2, 16)

                    def put(ref, val):
                        ref[slot] = jnp.where(lane0, val, ref[slot])

                    put(kx1, x1c)
                    put(ky1, y1c)
                    put(kx2, x2c)
                    put(ky2, y2c)
                    put(kar, ac)
                    put(ksel, idx_v[iw][0] - flat_base)
                    put(ksc, sc_v[iw][0])
                    put(kcl, cls_v[iw][0])

                return count2 + keep.astype(jnp.int32)

            scan_len = jnp.where(chunk_active, clen, 0)
            return lax.fori_loop(0, scan_len, cand_body, count)

        count = lax.fori_loop(0, nchunks, chunk_body, jnp.int32(0))

        meta_v[...] = jnp.where(lane0, count, 0)

        pltpu.sync_copy(ksel, sel_o.at[b])
        pltpu.sync_copy(ksc, osc_o.at[b])
        pltpu.sync_copy(kx1, ox1_o.at[b])
        pltpu.sync_copy(ky1, oy1_o.at[b])
        pltpu.sync_copy(kx2, ox2_o.at[b])
        pltpu.sync_copy(ky2, oy2_o.at[b])
        pltpu.sync_copy(kcl, ocl_o.at[b])
        pltpu.sync_copy(meta_v, meta_o.at[b])


def kernel(scores, boxes, classes):
    B, N = scores.shape
    Np = ((N + _CH - 1) // _CH) * _CH

    iota = jnp.broadcast_to(jnp.arange(N, dtype=jnp.int32)[None, :], (B, N))
    sneg, order = lax.sort((-scores, iota), dimension=1, num_keys=1,
                           is_stable=True)
    ssc = jnp.pad(-sneg, ((0, 0), (0, Np - N)))
    ord_off = order + (jnp.arange(B, dtype=jnp.int32) * N)[:, None]
    ordp = jnp.pad(ord_off, ((0, 0), (0, Np - N)))

    scf = ssc.reshape(-1)
    ordf = ordp.reshape(-1)
    x1f = boxes[:, :, 0].reshape(-1)
    y1f = boxes[:, :, 1].reshape(-1)
    x2f = boxes[:, :, 2].reshape(-1)
    y2f = boxes[:, :, 3].reshape(-1)
    clsf = classes.reshape(-1)

    mesh = plsc.VectorSubcoreMesh(core_axis_name="c", subcore_axis_name="s")
    out_type = [
        jax.ShapeDtypeStruct((B, _KP), jnp.int32),
        jax.ShapeDtypeStruct((B, _KP), jnp.float32),
        jax.ShapeDtypeStruct((B, _KP), jnp.float32),
        jax.ShapeDtypeStruct((B, _KP), jnp.float32),
        jax.ShapeDtypeStruct((B, _KP), jnp.float32),
        jax.ShapeDtypeStruct((B, _KP), jnp.float32),
        jax.ShapeDtypeStruct((B, _KP), jnp.int32),
        jax.ShapeDtypeStruct((B, 16), jnp.int32),
    ]
    scratch_types = [
        pltpu.VMEM((_CH + 16,), jnp.int32),
        pltpu.VMEM((_CH + 16,), jnp.float32),
        pltpu.VMEM((_CH + 16,), jnp.float32),
        pltpu.VMEM((_CH + 16,), jnp.float32),
        pltpu.VMEM((_CH + 16,), jnp.float32),
        pltpu.VMEM((_CH + 16,), jnp.float32),
        pltpu.VMEM((_CH + 16,), jnp.int32),
        pltpu.VMEM((_KP,), jnp.float32),
        pltpu.VMEM((_KP,), jnp.float32),
        pltpu.VMEM((_KP,), jnp.float32),
        pltpu.VMEM((_KP,), jnp.float32),
        pltpu.VMEM((_KP,), jnp.float32),
        pltpu.VMEM((_KP,), jnp.int32),
        pltpu.VMEM((_KP,), jnp.float32),
        pltpu.VMEM((_KP,), jnp.int32),
        pltpu.VMEM((16,), jnp.int32),
        pltpu.VMEM((16,), jnp.int32),
        pltpu.SemaphoreType.DMA,
    ]
    fn = pl.kernel(
        functools.partial(_nms_sc_body, N, Np, B),
        out_type=out_type,
        mesh=mesh,
        scratch_types=scratch_types,
    )
    sel_p, osc, ox1, oy1, ox2, oy2, ocl, meta = fn(
        scf, ordf, x1f, y1f, x2f, y2f, clsf)

    sel = sel_p[:, :_K]
    count = meta[:, 0]
    overflow = count > _K
    count = jnp.minimum(count, _K)
    eff = jnp.where(overflow, jnp.int32(_K - 1), count)
    m = jnp.arange(_K, dtype=jnp.int32)[None, :] < eff[:, None]
    out_scores = jnp.where(m, osc[:, :_K], 0.0)
    out_boxes = jnp.where(
        m[:, :, None],
        jnp.stack([ox1[:, :_K], oy1[:, :_K], ox2[:, :_K], oy2[:, :_K]],
                  axis=-1),
        0.0)
    out_classes = jnp.where(m, ocl[:, :_K], jnp.int32(_INT32_MAX))
    true_max = jnp.where(overflow, jnp.int32(-1), count).astype(jnp.int32)
    return (sel, out_scores, out_boxes, out_classes, true_max)


# speedup vs baseline: 7.4519x; 1.0827x over previous
import functools

import jax
import jax.numpy as jnp
from jax import lax
from jax.experimental import pallas as pl
from jax.experimental.pallas import tpu as pltpu
from jax.experimental.pallas import tpu_sc as plsc

_IOU_THRESHOLD = 0.5
_K = 300
_K1 = _K + 1
_KP = 320
_INT32_MAX = 2147483647
_CH = 1024
_ROWS = _CH // 128


def _take16(v, idx):
    dnums = lax.GatherDimensionNumbers(
        offset_dims=(), collapsed_slice_dims=(0,), start_index_map=(0,))
    return lax.gather(v, idx[:, None], dnums, slice_sizes=(1,),
                      mode=lax.GatherScatterMode.PROMISE_IN_BOUNDS)


def _nms_sc_body(n_scan, np_pad, n_stride, n_batches,
                 sc_hbm, ord_hbm, x1_hbm, y1_hbm, x2_hbm, y2_hbm, cls_hbm,
                 sel_o, osc_o, ox1_o, oy1_o, ox2_o, oy2_o, ocl_o, meta_o,
                 idx_v, sc_v, x1_v, y1_v, x2_v, y2_v, cls_v,
                 kx1, ky1, kx2, ky2, kar, ksel, ksc, kcl, meta_v, bad_v,
                 sem):
    cid = lax.axis_index("c")
    sid = lax.axis_index("s")
    b = sid * 2 + cid
    i16 = lax.broadcasted_iota(jnp.int32, (16,), 0)
    lane0 = i16 == 0

    @pl.when(b < n_batches)
    def _run():
        zf = jnp.zeros((16,), jnp.float32)
        neg1 = jnp.full((16,), -1, jnp.int32)
        zi = jnp.zeros((16,), jnp.int32)
        for j in range(_KP // 16):
            s = pl.ds(j * 16, 16)
            kx1[s] = zf
            ky1[s] = zf
            kx2[s] = zf
            ky2[s] = zf
            kar[s] = zf
            ksel[s] = neg1
            ksc[s] = zf
            kcl[s] = zi

        nchunks = (n_scan + _CH - 1) // _CH
        flat_base = b * n_stride
        pad_base = b * np_pad

        def chunk_body(ci, count):
            chunk_active = count < _K1
            c0 = ci * _CH
            clen = jnp.minimum(n_scan - c0, _CH)

            @pl.when(chunk_active)
            def _fetch():
                pltpu.sync_copy(sc_hbm.at[pl.ds(pad_base + c0, _CH)],
                                sc_v.at[pl.ds(0, _CH)])
                pltpu.sync_copy(ord_hbm.at[pl.ds(pad_base + c0, _CH)],
                                idx_v.at[pl.ds(0, _CH)])
                for j in range(_ROWS):
                    row = idx_v.at[pl.ds(j * 128, 128)]
                    dst = pl.ds(j * 128, 128)
                    cps = (pltpu.async_copy(x1_hbm.at[row], x1_v.at[dst], sem),
                           pltpu.async_copy(y1_hbm.at[row], y1_v.at[dst], sem),
                           pltpu.async_copy(x2_hbm.at[row], x2_v.at[dst], sem),
                           pltpu.async_copy(y2_hbm.at[row], y2_v.at[dst], sem),
                           pltpu.async_copy(cls_hbm.at[row], cls_v.at[dst],
                                            sem))
                    for cp in cps:
                        cp.wait()

            def cand_body(i, count2):
                active = count2 < _K1
                iw = pl.ds(i, 16)
                x1c = x1_v[iw][0]
                y1c = y1_v[iw][0]
                x2c = x2_v[iw][0]
                y2c = y2_v[iw][0]
                ac = (x2c - x1c) * (y2c - y1c)

                nkc = jnp.where(active, (count2 + 15) // 16, 0)
                bad_v[...] = jnp.zeros((16,), jnp.int32)

                def kchunk(jk, tok):
                    ks = pl.ds(jk * 16, 16)
                    xx1 = jnp.maximum(kx1[ks], x1c)
                    yy1 = jnp.maximum(ky1[ks], y1c)
                    xx2 = jnp.minimum(kx2[ks], x2c)
                    yy2 = jnp.minimum(ky2[ks], y2c)
                    inter = (jnp.maximum(xx2 - xx1, 0.0) *
                             jnp.maximum(yy2 - yy1, 0.0))
                    iou = inter / (kar[ks] + ac - inter + 1e-9)
                    bad_v[...] = bad_v[...] | jnp.where(
                        iou > _IOU_THRESHOLD, 1, 0)
                    return tok

                lax.fori_loop(0, nkc, kchunk, jnp.int32(0))
                v = bad_v[...]
                for sh in (8, 4, 2, 1):
                    v = jnp.maximum(v, _take16(v, (i16 + sh) & 15))
                keep = active & (v[0] == 0)

                @pl.when(keep & (count2 < _K))
                def _store():
                    slot = pl.ds(count2, 16)

                    def put(ref, val):
                        ref[slot] = jnp.where(lane0, val, ref[slot])

                    put(kx1, x1c)
                    put(ky1, y1c)
                    put(kx2, x2c)
                    put(ky2, y2c)
                    put(kar, ac)
                    put(ksel, idx_v[iw][0] - flat_base)
                    put(ksc, sc_v[iw][0])
                    put(kcl, cls_v[iw][0])

                return count2 + keep.astype(jnp.int32)

            scan_len = jnp.where(chunk_active, clen, 0)
            return lax.fori_loop(0, scan_len, cand_body, count)

        count = lax.fori_loop(0, nchunks, chunk_body, jnp.int32(0))

        meta_v[...] = jnp.where(lane0, count, 0)

        pltpu.sync_copy(ksel, sel_o.at[b])
        pltpu.sync_copy(ksc, osc_o.at[b])
        pltpu.sync_copy(kx1, ox1_o.at[b])
        pltpu.sync_copy(ky1, oy1_o.at[b])
        pltpu.sync_copy(kx2, ox2_o.at[b])
        pltpu.sync_copy(ky2, oy2_o.at[b])
        pltpu.sync_copy(kcl, ocl_o.at[b])
        pltpu.sync_copy(meta_v, meta_o.at[b])


_T_FAST = 2048


def kernel(scores, boxes, classes):
    B, N = scores.shape
    Np = ((N + _CH - 1) // _CH) * _CH

    x1f = boxes[:, :, 0].reshape(-1)
    y1f = boxes[:, :, 1].reshape(-1)
    x2f = boxes[:, :, 2].reshape(-1)
    y2f = boxes[:, :, 3].reshape(-1)
    clsf = classes.reshape(-1)
    boff = (jnp.arange(B, dtype=jnp.int32) * N)[:, None]

    def run_scan(ssc, ord_off, n_scan):
        return tuple(_scan_call(N, B, ssc.shape[1], n_scan)(
            ssc.reshape(-1), ord_off.reshape(-1),
            x1f, y1f, x2f, y2f, clsf))

    vals, _idxs = lax.top_k(scores, _T_FAST)
    sneg_f, idx_f = lax.sort((-vals, _idxs.astype(jnp.int32)),
                             dimension=1, num_keys=2)
    fast = run_scan(-sneg_f, idx_f + boff, _T_FAST)

    vmin = vals[:, -1:]
    cnt_full = jnp.sum((scores == vmin).astype(jnp.int32), axis=1)
    cnt_sel = jnp.sum((vals == vmin).astype(jnp.int32), axis=1)
    boundary_tie = jnp.any(cnt_full > cnt_sel)
    short = jnp.any(fast[7][:, 0] < _K1)
    need_full = boundary_tie | short

    def full_path():
        iota = jnp.broadcast_to(
            jnp.arange(Np, dtype=jnp.int32)[None, :], (B, Np))
        keys = jnp.pad(-scores, ((0, 0), (0, Np - N)),
                       constant_values=jnp.inf)
        sneg, order = lax.sort((keys, iota), dimension=1, num_keys=1,
                               is_stable=True)
        ord_off = jnp.minimum(order, N - 1) + boff
        return run_scan(-sneg, ord_off, N)

    sel_p, osc, ox1, oy1, ox2, oy2, ocl, meta = lax.cond(
        need_full, full_path, lambda: fast)

    sel = sel_p[:, :_K]
    count = meta[:, 0]
    overflow = count > _K
    count = jnp.minimum(count, _K)
    eff = jnp.where(overflow, jnp.int32(_K - 1), count)
    m = jnp.arange(_K, dtype=jnp.int32)[None, :] < eff[:, None]
    out_scores = jnp.where(m, osc[:, :_K], 0.0)
    out_boxes = jnp.where(
        m[:, :, None],
        jnp.stack([ox1[:, :_K], oy1[:, :_K], ox2[:, :_K], oy2[:, :_K]],
                  axis=-1),
        0.0)
    out_classes = jnp.where(m, ocl[:, :_K], jnp.int32(_INT32_MAX))
    true_max = jnp.where(overflow, jnp.int32(-1), count).astype(jnp.int32)
    return (sel, out_scores, out_boxes, out_classes, true_max)


def _scan_call(n_stride, n_batches, t_pad, n_scan):
    B = n_batches
    mesh = plsc.VectorSubcoreMesh(core_axis_name="c", subcore_axis_name="s")
    out_type = [
        jax.ShapeDtypeStruct((B, _KP), jnp.int32),
        jax.ShapeDtypeStruct((B, _KP), jnp.float32),
        jax.ShapeDtypeStruct((B, _KP), jnp.float32),
        jax.ShapeDtypeStruct((B, _KP), jnp.float32),
        jax.ShapeDtypeStruct((B, _KP), jnp.float32),
        jax.ShapeDtypeStruct((B, _KP), jnp.float32),
        jax.ShapeDtypeStruct((B, _KP), jnp.int32),
        jax.ShapeDtypeStruct((B, 16), jnp.int32),
    ]
    scratch_types = [
        pltpu.VMEM((_CH + 16,), jnp.int32),
        pltpu.VMEM((_CH + 16,), jnp.float32),
        pltpu.VMEM((_CH + 16,), jnp.float32),
        pltpu.VMEM((_CH + 16,), jnp.float32),
        pltpu.VMEM((_CH + 16,), jnp.float32),
        pltpu.VMEM((_CH + 16,), jnp.float32),
        pltpu.VMEM((_CH + 16,), jnp.int32),
        pltpu.VMEM((_KP,), jnp.float32),
        pltpu.VMEM((_KP,), jnp.float32),
        pltpu.VMEM((_KP,), jnp.float32),
        pltpu.VMEM((_KP,), jnp.float32),
        pltpu.VMEM((_KP,), jnp.float32),
        pltpu.VMEM((_KP,), jnp.int32),
        pltpu.VMEM((_KP,), jnp.float32),
        pltpu.VMEM((_KP,), jnp.int32),
        pltpu.VMEM((16,), jnp.int32),
        pltpu.VMEM((16,), jnp.int32),
        pltpu.SemaphoreType.DMA,
    ]
    return pl.kernel(
        functools.partial(_nms_sc_body, n_scan, t_pad, n_stride, n_batches),
        out_type=out_type,
        mesh=mesh,
        scratch_types=scratch_types,
    )


# speedup vs baseline: 8.0744x; 1.0835x over previous
import functools

import jax
import jax.numpy as jnp
from jax import lax
from jax.experimental import pallas as pl
from jax.experimental.pallas import tpu as pltpu
from jax.experimental.pallas import tpu_sc as plsc

_IOU_THRESHOLD = 0.5
_K = 300
_K1 = _K + 1
_KP = 320
_INT32_MAX = 2147483647
_CH = 1024
_ROWS = _CH // 128


def _take16(v, idx):
    dnums = lax.GatherDimensionNumbers(
        offset_dims=(), collapsed_slice_dims=(0,), start_index_map=(0,))
    return lax.gather(v, idx[:, None], dnums, slice_sizes=(1,),
                      mode=lax.GatherScatterMode.PROMISE_IN_BOUNDS)


def _nms_sc_body(n_scan, np_pad, n_stride, n_batches,
                 sc_hbm, ord_hbm, x1_hbm, y1_hbm, x2_hbm, y2_hbm, cls_hbm,
                 sel_o, osc_o, ox1_o, oy1_o, ox2_o, oy2_o, ocl_o, meta_o,
                 idx_v, sc_v, x1_v, y1_v, x2_v, y2_v, cls_v,
                 kx1, ky1, kx2, ky2, kar, ksel, ksc, kcl, meta_v, bad_v,
                 sem):
    cid = lax.axis_index("c")
    sid = lax.axis_index("s")
    b = sid * 2 + cid
    i16 = lax.broadcasted_iota(jnp.int32, (16,), 0)
    lane0 = i16 == 0

    @pl.when(b < n_batches)
    def _run():
        zf = jnp.zeros((16,), jnp.float32)
        neg1 = jnp.full((16,), -1, jnp.int32)
        zi = jnp.zeros((16,), jnp.int32)
        for j in range(_KP // 16):
            s = pl.ds(j * 16, 16)
            kx1[s] = zf
            ky1[s] = zf
            kx2[s] = zf
            ky2[s] = zf
            kar[s] = zf
            ksel[s] = neg1
            ksc[s] = zf
            kcl[s] = zi

        nchunks = (n_scan + _CH - 1) // _CH
        flat_base = b * n_stride
        pad_base = b * np_pad

        def chunk_body(ci, count):
            chunk_active = count < _K1
            c0 = ci * _CH
            clen = jnp.minimum(n_scan - c0, _CH)

            @pl.when(chunk_active)
            def _fetch():
                pltpu.sync_copy(sc_hbm.at[pl.ds(pad_base + c0, _CH)],
                                sc_v.at[pl.ds(0, _CH)])
                pltpu.sync_copy(ord_hbm.at[pl.ds(pad_base + c0, _CH)],
                                idx_v.at[pl.ds(0, _CH)])
                for j in range(_ROWS):
                    row = idx_v.at[pl.ds(j * 128, 128)]
                    dst = pl.ds(j * 128, 128)
                    cps = (pltpu.async_copy(x1_hbm.at[row], x1_v.at[dst], sem),
                           pltpu.async_copy(y1_hbm.at[row], y1_v.at[dst], sem),
                           pltpu.async_copy(x2_hbm.at[row], x2_v.at[dst], sem),
                           pltpu.async_copy(y2_hbm.at[row], y2_v.at[dst], sem),
                           pltpu.async_copy(cls_hbm.at[row], cls_v.at[dst],
                                            sem))
                    for cp in cps:
                        cp.wait()

            def cand_body(i, count2):
                active = count2 < _K1
                iw = pl.ds(i, 16)
                x1c = x1_v[iw][0]
                y1c = y1_v[iw][0]
                x2c = x2_v[iw][0]
                y2c = y2_v[iw][0]
                ac = (x2c - x1c) * (y2c - y1c)

                nkc = jnp.where(active, (count2 + 63) // 64, 0)
                bad_v[...] = jnp.zeros((16,), jnp.float32)

                def kchunk(jk, tok):
                    worst = jnp.zeros((16,), jnp.float32)
                    for g in range(4):
                        ks = pl.ds(jk * 64 + g * 16, 16)
                        xx1 = jnp.maximum(kx1[ks], x1c)
                        yy1 = jnp.maximum(ky1[ks], y1c)
                        xx2 = jnp.minimum(kx2[ks], x2c)
                        yy2 = jnp.minimum(ky2[ks], y2c)
                        inter = (jnp.maximum(xx2 - xx1, 0.0) *
                                 jnp.maximum(yy2 - yy1, 0.0))
                        iou = inter / (kar[ks] + ac - inter + 1e-9)
                        worst = jnp.maximum(worst, iou)
                    bad_v[...] = jnp.maximum(bad_v[...], worst)
                    return tok

                lax.fori_loop(0, nkc, kchunk, jnp.int32(0))
                v = bad_v[...]
                for sh in (8, 4, 2, 1):
                    v = jnp.maximum(v, _take16(v, (i16 + sh) & 15))
                keep = active & (v[0] <= _IOU_THRESHOLD)

                @pl.when(keep & (count2 < _K))
                def _store():
                    slot = pl.ds(count2, 16)

                    def put(ref, val):
                        ref[slot] = jnp.where(lane0, val, ref[slot])

                    put(kx1, x1c)
                    put(ky1, y1c)
                    put(kx2, x2c)
                    put(ky2, y2c)
                    put(kar, ac)
                    put(ksel, idx_v[iw][0] - flat_base)
                    put(ksc, sc_v[iw][0])
                    put(kcl, cls_v[iw][0])

                return count2 + keep.astype(jnp.int32)

            scan_len = jnp.where(chunk_active, clen, 0)
            return lax.fori_loop(0, scan_len, cand_body, count)

        count = lax.fori_loop(0, nchunks, chunk_body, jnp.int32(0))

        meta_v[...] = jnp.where(lane0, count, 0)

        pltpu.sync_copy(ksel, sel_o.at[b])
        pltpu.sync_copy(ksc, osc_o.at[b])
        pltpu.sync_copy(kx1, ox1_o.at[b])
        pltpu.sync_copy(ky1, oy1_o.at[b])
        pltpu.sync_copy(kx2, ox2_o.at[b])
        pltpu.sync_copy(ky2, oy2_o.at[b])
        pltpu.sync_copy(kcl, ocl_o.at[b])
        pltpu.sync_copy(meta_v, meta_o.at[b])


_T_FAST = 2048


def kernel(scores, boxes, classes):
    B, N = scores.shape
    Np = ((N + _CH - 1) // _CH) * _CH

    x1f = boxes[:, :, 0].reshape(-1)
    y1f = boxes[:, :, 1].reshape(-1)
    x2f = boxes[:, :, 2].reshape(-1)
    y2f = boxes[:, :, 3].reshape(-1)
    clsf = classes.reshape(-1)
    boff = (jnp.arange(B, dtype=jnp.int32) * N)[:, None]

    def run_scan(ssc, ord_off, n_scan):
        return tuple(_scan_call(N, B, ssc.shape[1], n_scan)(
            ssc.reshape(-1), ord_off.reshape(-1),
            x1f, y1f, x2f, y2f, clsf))

    vals, _idxs = lax.top_k(scores, _T_FAST)
    sneg_f, idx_f = lax.sort((-vals, _idxs.astype(jnp.int32)),
                             dimension=1, num_keys=2)
    fast = run_scan(-sneg_f, idx_f + boff, _T_FAST)

    vmin = vals[:, -1:]
    cnt_full = jnp.sum((scores == vmin).astype(jnp.int32), axis=1)
    cnt_sel = jnp.sum((vals == vmin).astype(jnp.int32), axis=1)
    boundary_tie = jnp.any(cnt_full > cnt_sel)
    short = jnp.any(fast[7][:, 0] < _K1)
    need_full = boundary_tie | short

    def full_path():
        iota = jnp.broadcast_to(
            jnp.arange(Np, dtype=jnp.int32)[None, :], (B, Np))
        keys = jnp.pad(-scores, ((0, 0), (0, Np - N)),
                       constant_values=jnp.inf)
        sneg, order = lax.sort((keys, iota), dimension=1, num_keys=1,
                               is_stable=True)
        ord_off = jnp.minimum(order, N - 1) + boff
        return run_scan(-sneg, ord_off, N)

    sel_p, osc, ox1, oy1, ox2, oy2, ocl, meta = lax.cond(
        need_full, full_path, lambda: fast)

    sel = sel_p[:, :_K]
    count = meta[:, 0]
    overflow = count > _K
    count = jnp.minimum(count, _K)
    eff = jnp.where(overflow, jnp.int32(_K - 1), count)
    m = jnp.arange(_K, dtype=jnp.int32)[None, :] < eff[:, None]
    out_scores = jnp.where(m, osc[:, :_K], 0.0)
    out_boxes = jnp.where(
        m[:, :, None],
        jnp.stack([ox1[:, :_K], oy1[:, :_K], ox2[:, :_K], oy2[:, :_K]],
                  axis=-1),
        0.0)
    out_classes = jnp.where(m, ocl[:, :_K], jnp.int32(_INT32_MAX))
    true_max = jnp.where(overflow, jnp.int32(-1), count).astype(jnp.int32)
    return (sel, out_scores, out_boxes, out_classes, true_max)


def _scan_call(n_stride, n_batches, t_pad, n_scan):
    B = n_batches
    mesh = plsc.VectorSubcoreMesh(core_axis_name="c", subcore_axis_name="s")
    out_type = [
        jax.ShapeDtypeStruct((B, _KP), jnp.int32),
        jax.ShapeDtypeStruct((B, _KP), jnp.float32),
        jax.ShapeDtypeStruct((B, _KP), jnp.float32),
        jax.ShapeDtypeStruct((B, _KP), jnp.float32),
        jax.ShapeDtypeStruct((B, _KP), jnp.float32),
        jax.ShapeDtypeStruct((B, _KP), jnp.float32),
        jax.ShapeDtypeStruct((B, _KP), jnp.int32),
        jax.ShapeDtypeStruct((B, 16), jnp.int32),
    ]
    scratch_types = [
        pltpu.VMEM((_CH + 16,), jnp.int32),
        pltpu.VMEM((_CH + 16,), jnp.float32),
        pltpu.VMEM((_CH + 16,), jnp.float32),
        pltpu.VMEM((_CH + 16,), jnp.float32),
        pltpu.VMEM((_CH + 16,), jnp.float32),
        pltpu.VMEM((_CH + 16,), jnp.float32),
        pltpu.VMEM((_CH + 16,), jnp.int32),
        pltpu.VMEM((_KP,), jnp.float32),
        pltpu.VMEM((_KP,), jnp.float32),
        pltpu.VMEM((_KP,), jnp.float32),
        pltpu.VMEM((_KP,), jnp.float32),
        pltpu.VMEM((_KP,), jnp.float32),
        pltpu.VMEM((_KP,), jnp.int32),
        pltpu.VMEM((_KP,), jnp.float32),
        pltpu.VMEM((_KP,), jnp.int32),
        pltpu.VMEM((16,), jnp.int32),
        pltpu.VMEM((16,), jnp.float32),
        pltpu.SemaphoreType.DMA,
    ]
    return pl.kernel(
        functools.partial(_nms_sc_body, n_scan, t_pad, n_stride, n_batches),
        out_type=out_type,
        mesh=mesh,
        scratch_types=scratch_types,
    )


# speedup vs baseline: 24.0020x; 2.9726x over previous
import functools

import jax
import jax.numpy as jnp
from jax import lax
from jax.experimental import pallas as pl
from jax.experimental.pallas import tpu as pltpu
from jax.experimental.pallas import tpu_sc as plsc

_IOU_THRESHOLD = 0.5
_K = 300
_K1 = _K + 1
_KP = 384
_INT32_MAX = 2147483647


def _take16(v, idx):
    dnums = lax.GatherDimensionNumbers(
        offset_dims=(), collapsed_slice_dims=(0,), start_index_map=(0,))
    return lax.gather(v, idx[:, None], dnums, slice_sizes=(1,),
                      mode=lax.GatherScatterMode.PROMISE_IN_BOUNDS)


def _nms_body(n_boxes, np_pad, n_batches,
              sc_hbm, x1_hbm, y1_hbm, x2_hbm, y2_hbm, cls_hbm,
              sel_o, osc_o, ox1_o, oy1_o, ox2_o, oy2_o, ocl_o, meta_o,
              sc0, sx1, sy1, sx2, sy2, l1, l2,
              kx1, ky1, kx2, ky2, kar, ksel, ksc, kcl, gidx,
              meta_v, bad_v, st_s, sem):
    cid = lax.axis_index("c")
    sid = lax.axis_index("s")
    b = sid * 2 + cid
    i16 = lax.broadcasted_iota(jnp.int32, (16,), 0)
    lane0 = i16 == 0
    neg_inf = jnp.float32(-jnp.inf)
    nb1 = np_pad // 16
    nb2 = nb1 // 16

    def smax(v):
        for sh in (8, 4, 2, 1):
            v = jnp.maximum(v, _take16(v, (i16 + sh) & 15))
        return v

    def smin(v):
        for sh in (8, 4, 2, 1):
            v = jnp.minimum(v, _take16(v, (i16 + sh) & 15))
        return v

    def put(ref, pos, val):
        w = pl.ds(pos, 16)
        ref[w] = jnp.where(lane0, val, ref[w])

    @pl.when(b < n_batches)
    def _run():
        base = b * n_boxes
        zf = jnp.zeros((16,), jnp.float32)
        ninf = jnp.full((16,), neg_inf)

        cps = (pltpu.async_copy(sc_hbm.at[pl.ds(base, n_boxes)],
                                sc0.at[pl.ds(0, n_boxes)], sem),
               pltpu.async_copy(x1_hbm.at[pl.ds(base, n_boxes)],
                                sx1.at[pl.ds(0, n_boxes)], sem),
               pltpu.async_copy(y1_hbm.at[pl.ds(base, n_boxes)],
                                sy1.at[pl.ds(0, n_boxes)], sem),
               pltpu.async_copy(x2_hbm.at[pl.ds(base, n_boxes)],
                                sx2.at[pl.ds(0, n_boxes)], sem),
               pltpu.async_copy(y2_hbm.at[pl.ds(base, n_boxes)],
                                sy2.at[pl.ds(0, n_boxes)], sem))

        for j in range(_KP // 16):
            s = pl.ds(j * 16, 16)
            kx1[s] = zf
            ky1[s] = zf
            kx2[s] = zf
            ky2[s] = zf
            kar[s] = zf
            ksel[s] = jnp.full((16,), -1, jnp.int32)
            ksc[s] = zf
        for cp in cps:
            cp.wait()
        for j in range((np_pad - n_boxes) // 16):
            sc0[pl.ds(n_boxes + j * 16, 16)] = ninf

        def b1_build(jk, tok):
            v = sc0[pl.ds(jk * 16, 16)]
            put(l1, jk, smax(v)[0])
            return tok

        lax.fori_loop(0, nb1, b1_build, jnp.int32(0))

        def b2_build(jk, tok):
            v = l1[pl.ds(jk * 16, 16)]
            put(l2, jk, smax(v)[0])
            return tok

        lax.fori_loop(0, nb2, b2_build, jnp.int32(0))

        st_s[0] = 0
        st_s[1] = 0

        big = jnp.full((16,), 9999, jnp.int32)

        def examine(_, tok):
            count = st_s[0]
            active = (count < _K1) & (st_s[1] == 0)

            @pl.when(active)
            def _one():
                vs = [l2[pl.ds(16 * j, 16)] for j in range(nb2 // 16)]
                m = vs[0]
                for v in vs[1:]:
                    m = jnp.maximum(m, v)
                gm = smax(m)[0]

                @pl.when(gm == neg_inf)
                def _done():
                    st_s[1] = 1

                @pl.when(gm > neg_inf)
                def _pick():
                    cand = big
                    for j, v in enumerate(vs):
                        cand = jnp.minimum(
                            cand, jnp.where(v == gm, i16 + 16 * j, 9999))
                    b2i = smin(cand)[0]
                    w1 = l1[pl.ds(b2i * 16, 16)]
                    j1 = smin(jnp.where(w1 == gm, i16, 9999))[0]
                    b1i = b2i * 16 + j1
                    s0 = sc0[pl.ds(b1i * 16, 16)]
                    l0 = smin(jnp.where(s0 == gm, i16, 9999))[0]
                    idx = b1i * 16 + l0

                    iw = pl.ds(idx, 16)
                    x1c = sx1[iw][0]
                    y1c = sy1[iw][0]
                    x2c = sx2[iw][0]
                    y2c = sy2[iw][0]
                    ac = (x2c - x1c) * (y2c - y1c)

                    nkc = (count + 63) // 64
                    bad_v[...] = zf

                    def kchunk(jk, tok2):
                        worst = zf
                        for g in range(4):
                            ks = pl.ds(jk * 64 + g * 16, 16)
                            xx1 = jnp.maximum(kx1[ks], x1c)
                            yy1 = jnp.maximum(ky1[ks], y1c)
                            xx2 = jnp.minimum(kx2[ks], x2c)
                            yy2 = jnp.minimum(ky2[ks], y2c)
                            inter = (jnp.maximum(xx2 - xx1, 0.0) *
                                     jnp.maximum(yy2 - yy1, 0.0))
                            iou = inter / (kar[ks] + ac - inter + 1e-9)
                            worst = jnp.maximum(worst, iou)
                        bad_v[...] = jnp.maximum(bad_v[...], worst)
                        return tok2

                    lax.fori_loop(0, nkc, kchunk, jnp.int32(0))
                    wv = smax(bad_v[...])
                    keep = wv[0] <= _IOU_THRESHOLD

                    @pl.when(keep & (count < _K))
                    def _store():
                        put(kx1, count, x1c)
                        put(ky1, count, y1c)
                        put(kx2, count, x2c)
                        put(ky2, count, y2c)
                        put(kar, count, ac)
                        put(ksel, count, idx)
                        put(ksc, count, gm)

                    st_s[0] = count + keep.astype(jnp.int32)

                    s0n = jnp.where(i16 == l0, neg_inf, s0)
                    sc0[pl.ds(b1i * 16, 16)] = s0n
                    put(l1, b1i, smax(s0n)[0])
                    w1n = l1[pl.ds(b2i * 16, 16)]
                    put(l2, b2i, smax(w1n)[0])

            return tok

        def outer(_, tok):
            @pl.when((st_s[0] < _K1) & (st_s[1] == 0))
            def _block():
                lax.fori_loop(0, 128, examine, jnp.int32(0))
            return tok

        lax.fori_loop(0, np_pad // 128, outer, jnp.int32(0))

        count = st_s[0]
        meta_v[...] = jnp.where(lane0, count, 0)

        for j in range(_KP // 16):
            s = pl.ds(j * 16, 16)
            gidx[s] = jnp.maximum(ksel[s], 0) + base
        gcps = []
        for j in range(_KP // 128):
            row = gidx.at[pl.ds(j * 128, 128)]
            gcps.append(pltpu.async_copy(cls_hbm.at[row],
                                         kcl.at[pl.ds(j * 128, 128)], sem))
        for cp in gcps:
            cp.wait()

        pltpu.sync_copy(ksel, sel_o.at[b])
        pltpu.sync_copy(ksc, osc_o.at[b])
        pltpu.sync_copy(kx1, ox1_o.at[b])
        pltpu.sync_copy(ky1, oy1_o.at[b])
        pltpu.sync_copy(kx2, ox2_o.at[b])
        pltpu.sync_copy(ky2, oy2_o.at[b])
        pltpu.sync_copy(kcl, ocl_o.at[b])
        pltpu.sync_copy(meta_v, meta_o.at[b])


def kernel(scores, boxes, classes):
    B, N = scores.shape
    Np = ((N + 2047) // 2048) * 2048

    scf = scores.reshape(-1)
    x1f = boxes[:, :, 0].reshape(-1)
    y1f = boxes[:, :, 1].reshape(-1)
    x2f = boxes[:, :, 2].reshape(-1)
    y2f = boxes[:, :, 3].reshape(-1)
    clsf = classes.reshape(-1)

    mesh = plsc.VectorSubcoreMesh(core_axis_name="c", subcore_axis_name="s")
    out_type = [
        jax.ShapeDtypeStruct((B, _KP), jnp.int32),
        jax.ShapeDtypeStruct((B, _KP), jnp.float32),
        jax.ShapeDtypeStruct((B, _KP), jnp.float32),
        jax.ShapeDtypeStruct((B, _KP), jnp.float32),
        jax.ShapeDtypeStruct((B, _KP), jnp.float32),
        jax.ShapeDtypeStruct((B, _KP), jnp.float32),
        jax.ShapeDtypeStruct((B, _KP), jnp.int32),
        jax.ShapeDtypeStruct((B, 16), jnp.int32),
    ]
    scratch_types = [
        pltpu.VMEM((Np + 16,), jnp.float32),
        pltpu.VMEM((Np + 16,), jnp.float32),
        pltpu.VMEM((Np + 16,), jnp.float32),
        pltpu.VMEM((Np + 16,), jnp.float32),
        pltpu.VMEM((Np + 16,), jnp.float32),
        pltpu.VMEM((Np // 16 + 16,), jnp.float32),
        pltpu.VMEM((Np // 256 + 16,), jnp.float32),
        pltpu.VMEM((_KP,), jnp.float32),
        pltpu.VMEM((_KP,), jnp.float32),
        pltpu.VMEM((_KP,), jnp.float32),
        pltpu.VMEM((_KP,), jnp.float32),
        pltpu.VMEM((_KP,), jnp.float32),
        pltpu.VMEM((_KP,), jnp.int32),
        pltpu.VMEM((_KP,), jnp.float32),
        pltpu.VMEM((_KP,), jnp.int32),
        pltpu.VMEM((_KP,), jnp.int32),
        pltpu.VMEM((16,), jnp.int32),
        pltpu.VMEM((16,), jnp.float32),
        pltpu.SMEM((2,), jnp.int32),
        pltpu.SemaphoreType.DMA,
    ]
    fn = pl.kernel(
        functools.partial(_nms_body, N, Np, B),
        out_type=out_type,
        mesh=mesh,
        scratch_types=scratch_types,
    )
    sel_p, osc, ox1, oy1, ox2, oy2, ocl, meta = fn(
        scf, x1f, y1f, x2f, y2f, clsf)

    sel = sel_p[:, :_K]
    count = meta[:, 0]
    overflow = count > _K
    count = jnp.minimum(count, _K)
    eff = jnp.where(overflow, jnp.int32(_K - 1), count)
    m = jnp.arange(_K, dtype=jnp.int32)[None, :] < eff[:, None]
    out_scores = jnp.where(m, osc[:, :_K], 0.0)
    out_boxes = jnp.where(
        m[:, :, None],
        jnp.stack([ox1[:, :_K], oy1[:, :_K], ox2[:, :_K], oy2[:, :_K]],
                  axis=-1),
        0.0)
    out_classes = jnp.where(m, ocl[:, :_K], jnp.int32(_INT32_MAX))
    true_max = jnp.where(overflow, jnp.int32(-1), count).astype(jnp.int32)
    return (sel, out_scores, out_boxes, out_classes, true_max)
